# Initial kernel scaffold; baseline (speedup 1.0000x reference)
import jax
import jax.numpy as jnp
from jax.experimental import pallas as pl


def kernel(node_features, node_hidden, edge_index, obj_W1, obj_b1, obj_W2, obj_b2, obj_W3, obj_b3, nc_W1, nc_b1, nc_W2, nc_b2, nc_W3, nc_b3):
    raise NotImplementedError("write your pallas kernel here")


# speedup vs baseline: 2.9319x; 2.9319x over previous
import functools

import jax
import jax.numpy as jnp
from jax import lax
from jax.experimental import pallas as pl
from jax.experimental.pallas import tpu as pltpu
from jax.experimental.pallas import tpu_sc as plsc

N = 10000
E = 320000
D_FEAT = 128
D_HID = 128
H = 256
N_OBJ = 32
N_CLS = 8

ROW_BLK = 1000

SC_CORES = 2
SC_SUBCORES = 16
NW = SC_CORES * SC_SUBCORES
PER_W = E // NW
CHUNK = 80
GROUPS = CHUNK // 16


def _mean_body(hid_ref, out_ref):
    i = pl.program_id(0)

    @pl.when(i == 0)
    def _init():
        out_ref[...] = jnp.zeros_like(out_ref)

    out_ref[...] += jnp.sum(hid_ref[...], axis=0, keepdims=True)

    @pl.when(i == pl.num_programs(0) - 1)
    def _fin():
        out_ref[...] = out_ref[...] * (1.0 / N)


def _col_mean(node_hidden):
    return pl.pallas_call(
        _mean_body,
        grid=(N // ROW_BLK,),
        in_specs=[pl.BlockSpec((ROW_BLK, D_HID), lambda i: (i, 0))],
        out_specs=pl.BlockSpec((1, D_HID), lambda i: (0, 0)),
        out_shape=jax.ShapeDtypeStruct((1, D_HID), jnp.float32),
    )(node_hidden)


def _mlp_body(feat_ref, hid_ref, mean_ref,
              oW1, ob1, oW2, ob2, oW3, ob3,
              nW1, nb1, nW2, nb2, nW3, nb3,
              obj_ref, cls_ref):
    x = jnp.concatenate(
        [feat_ref[...], hid_ref[...],
         jnp.broadcast_to(mean_ref[...], (ROW_BLK, D_HID))], axis=1)
    h = jnp.dot(x, oW1[...], preferred_element_type=jnp.float32) + ob1[...]
    h = jnp.maximum(
        jnp.dot(h, oW2[...], preferred_element_type=jnp.float32) + ob2[...], 0.0)
    obj_ref[...] = jnp.maximum(
        jnp.dot(h, oW3[...], preferred_element_type=jnp.float32) + ob3[...], 0.0)
    g = jnp.dot(x, nW1[...], preferred_element_type=jnp.float32) + nb1[...]
    g = jnp.maximum(
        jnp.dot(g, nW2[...], preferred_element_type=jnp.float32) + nb2[...], 0.0)
    cls_ref[...] = jnp.dot(g, nW3[...], preferred_element_type=jnp.float32) + nb3[...]


def _full(shape):
    return pl.BlockSpec(shape, lambda i: tuple(0 for _ in shape))


def _mlps(node_features, node_hidden, mean_h,
          oW1, ob1, oW2, ob2, oW3, ob3,
          nW1, nb1, nW2, nb2, nW3, nb3):
    return pl.pallas_call(
        _mlp_body,
        grid=(N // ROW_BLK,),
        in_specs=[
            pl.BlockSpec((ROW_BLK, D_FEAT), lambda i: (i, 0)),
            pl.BlockSpec((ROW_BLK, D_HID), lambda i: (i, 0)),
            _full((1, D_HID)),
            _full((D_FEAT + 2 * D_HID, H)), _full((1, H)),
            _full((H, H)), _full((1, H)),
            _full((H, N_OBJ)), _full((1, N_OBJ)),
            _full((D_FEAT + 2 * D_HID, H)), _full((1, H)),
            _full((H, H)), _full((1, H)),
            _full((H, N_CLS)), _full((1, N_CLS)),
        ],
        out_specs=[
            pl.BlockSpec((ROW_BLK, N_OBJ), lambda i: (i, 0)),
            pl.BlockSpec((ROW_BLK, N_CLS), lambda i: (i, 0)),
        ],
        out_shape=[
            jax.ShapeDtypeStruct((N, N_OBJ), jnp.float32),
            jax.ShapeDtypeStruct((N, N_CLS), jnp.float32),
        ],
    )(node_features, node_hidden, mean_h,
      oW1, ob1, oW2, ob2, oW3, ob3,
      nW1, nb1, nW2, nb2, nW3, nb3)


def _edge_body(table_hbm, src_hbm, dst_hbm, out_hbm,
               idx_s, idx_d, rows_s, rows_d, out_v, sem_s, sem_d):
    wid = lax.axis_index("s") * SC_CORES + lax.axis_index("c")
    base = wid * PER_W
    pltpu.sync_copy(src_hbm.at[pl.ds(base, PER_W)], idx_s)
    pltpu.sync_copy(dst_hbm.at[pl.ds(base, PER_W)], idx_d)
    lane = lax.iota(jnp.int32, 16)

    def chunk_body(c, carry):
        off = c * CHUNK
        cp_s = pltpu.make_async_copy(
            table_hbm.at[idx_s.at[pl.ds(off, CHUNK)]], rows_s, sem_s)
        cp_d = pltpu.make_async_copy(
            table_hbm.at[idx_d.at[pl.ds(off, CHUNK)]], rows_d, sem_d)
        cp_s.start()
        cp_d.start()
        cp_s.wait()
        cp_d.wait()
        for g in range(GROUPS):
            row_ids = g * 16 + lane
            acc = jnp.zeros((16,), jnp.float32)
            for k in range(N_OBJ):
                col = jnp.full((16,), k, jnp.int32)
                sv = plsc.load_gather(rows_s, [row_ids, col])
                dv = plsc.load_gather(rows_d, [row_ids, col])
                acc = acc + sv * dv
            out_v[pl.ds(off + g * 16, 16)] = acc
        return carry

    lax.fori_loop(0, PER_W // CHUNK, chunk_body, 0)
    pltpu.sync_copy(out_v, out_hbm.at[pl.ds(base, PER_W)])


@functools.cache
def _edge_scores():
    return pl.kernel(
        _edge_body,
        out_type=jax.ShapeDtypeStruct((E,), jnp.float32),
        mesh=plsc.VectorSubcoreMesh(
            core_axis_name="c", subcore_axis_name="s",
            num_cores=SC_CORES, num_subcores=SC_SUBCORES),
        compiler_params=pltpu.CompilerParams(
            needs_layout_passes=False, use_tc_tiling_on_sc=False),
        scratch_types=[
            pltpu.VMEM((PER_W,), jnp.int32),
            pltpu.VMEM((PER_W,), jnp.int32),
            pltpu.VMEM((CHUNK, N_OBJ), jnp.float32),
            pltpu.VMEM((CHUNK, N_OBJ), jnp.float32),
            pltpu.VMEM((PER_W,), jnp.float32),
            pltpu.SemaphoreType.DMA,
            pltpu.SemaphoreType.DMA,
        ],
    )


def kernel(node_features, node_hidden, edge_index,
           obj_W1, obj_b1, obj_W2, obj_b2, obj_W3, obj_b3,
           nc_W1, nc_b1, nc_W2, nc_b2, nc_W3, nc_b3):
    mean_h = _col_mean(node_hidden)
    obj_pred, node_pred = _mlps(
        node_features, node_hidden, mean_h,
        obj_W1, obj_b1.reshape(1, H), obj_W2, obj_b2.reshape(1, H),
        obj_W3, obj_b3.reshape(1, N_OBJ),
        nc_W1, nc_b1.reshape(1, H), nc_W2, nc_b2.reshape(1, H),
        nc_W3, nc_b3.reshape(1, N_CLS))
    src = edge_index[0].astype(jnp.int32)
    dst = edge_index[1].astype(jnp.int32)
    edge_pred = _edge_scores()(obj_pred, src, dst)
    return obj_pred, edge_pred, node_pred


# speedup vs baseline: 4.1555x; 1.4173x over previous
import functools

import jax
import jax.numpy as jnp
from jax import lax
from jax.experimental import pallas as pl
from jax.experimental.pallas import tpu as pltpu
from jax.experimental.pallas import tpu_sc as plsc

N = 10000
E = 320000
D_FEAT = 128
D_HID = 128
H = 256
N_OBJ = 32
N_CLS = 8

ROW_BLK = 1000

SC_CORES = 2
SC_SUBCORES = 16
NW = SC_CORES * SC_SUBCORES
PER_W = E // NW
CHUNK = 400
GROUPS = CHUNK // 16
NCH = PER_W // CHUNK


def _mean_body(hid_ref, out_ref):
    i = pl.program_id(0)

    @pl.when(i == 0)
    def _init():
        out_ref[...] = jnp.zeros_like(out_ref)

    out_ref[...] += jnp.sum(hid_ref[...], axis=0, keepdims=True)

    @pl.when(i == pl.num_programs(0) - 1)
    def _fin():
        out_ref[...] = out_ref[...] * (1.0 / N)


def _col_mean(node_hidden):
    return pl.pallas_call(
        _mean_body,
        grid=(N // ROW_BLK,),
        in_specs=[pl.BlockSpec((ROW_BLK, D_HID), lambda i: (i, 0))],
        out_specs=pl.BlockSpec((1, D_HID), lambda i: (0, 0)),
        out_shape=jax.ShapeDtypeStruct((1, D_HID), jnp.float32),
    )(node_hidden)


def _mlp_body(feat_ref, hid_ref, mean_ref,
              oW1, ob1, oW2, ob2, oW3, ob3,
              nW1, nb1, nW2, nb2, nW3, nb3,
              obj_ref, cls_ref):
    x = jnp.concatenate(
        [feat_ref[...], hid_ref[...],
         jnp.broadcast_to(mean_ref[...], (ROW_BLK, D_HID))], axis=1)
    h = jnp.dot(x, oW1[...], preferred_element_type=jnp.float32) + ob1[...]
    h = jnp.maximum(
        jnp.dot(h, oW2[...], preferred_element_type=jnp.float32) + ob2[...], 0.0)
    obj_ref[...] = jnp.maximum(
        jnp.dot(h, oW3[...], preferred_element_type=jnp.float32) + ob3[...], 0.0)
    g = jnp.dot(x, nW1[...], preferred_element_type=jnp.float32) + nb1[...]
    g = jnp.maximum(
        jnp.dot(g, nW2[...], preferred_element_type=jnp.float32) + nb2[...], 0.0)
    cls_ref[...] = jnp.dot(g, nW3[...], preferred_element_type=jnp.float32) + nb3[...]


def _full(shape):
    return pl.BlockSpec(shape, lambda i: tuple(0 for _ in shape))


def _mlps(node_features, node_hidden, mean_h,
          oW1, ob1, oW2, ob2, oW3, ob3,
          nW1, nb1, nW2, nb2, nW3, nb3):
    return pl.pallas_call(
        _mlp_body,
        grid=(N // ROW_BLK,),
        in_specs=[
            pl.BlockSpec((ROW_BLK, D_FEAT), lambda i: (i, 0)),
            pl.BlockSpec((ROW_BLK, D_HID), lambda i: (i, 0)),
            _full((1, D_HID)),
            _full((D_FEAT + 2 * D_HID, H)), _full((1, H)),
            _full((H, H)), _full((1, H)),
            _full((H, N_OBJ)), _full((1, N_OBJ)),
            _full((D_FEAT + 2 * D_HID, H)), _full((1, H)),
            _full((H, H)), _full((1, H)),
            _full((H, N_CLS)), _full((1, N_CLS)),
        ],
        out_specs=[
            pl.BlockSpec((ROW_BLK, N_OBJ), lambda i: (i, 0)),
            pl.BlockSpec((ROW_BLK, N_CLS), lambda i: (i, 0)),
        ],
        out_shape=[
            jax.ShapeDtypeStruct((N, N_OBJ), jnp.float32),
            jax.ShapeDtypeStruct((N, N_CLS), jnp.float32),
        ],
    )(node_features, node_hidden, mean_h,
      oW1, ob1, oW2, ob2, oW3, ob3,
      nW1, nb1, nW2, nb2, nW3, nb3)


def _edge_body(table_hbm, src_hbm, dst_hbm, out_hbm,
               idx_s, idx_d, rs_a, rd_a, rs_b, rd_b, out_v,
               ss_a, sd_a, ss_b, sd_b):
    wid = lax.axis_index("s") * SC_CORES + lax.axis_index("c")
    base = wid * PER_W
    pltpu.sync_copy(src_hbm.at[pl.ds(base, PER_W)], idx_s)
    pltpu.sync_copy(dst_hbm.at[pl.ds(base, PER_W)], idx_d)
    lane = lax.iota(jnp.int32, 16)

    def start(c, rs, rd, ss, sd):
        off = c * CHUNK
        pltpu.make_async_copy(
            table_hbm.at[idx_s.at[pl.ds(off, CHUNK)]], rs, ss).start()
        pltpu.make_async_copy(
            table_hbm.at[idx_d.at[pl.ds(off, CHUNK)]], rd, sd).start()

    def wait(rs, rd, ss, sd):
        pltpu.make_async_copy(
            table_hbm.at[idx_s.at[pl.ds(0, CHUNK)]], rs, ss).wait()
        pltpu.make_async_copy(
            table_hbm.at[idx_d.at[pl.ds(0, CHUNK)]], rd, sd).wait()

    def compute(c, rs, rd):
        def group_body(g, carry):
            row_ids = g * 16 + lane
            acc = jnp.zeros((16,), jnp.float32)
            for k in range(N_OBJ):
                col = jnp.full((16,), k, jnp.int32)
                acc = acc + (plsc.load_gather(rs, [row_ids, col]) *
                             plsc.load_gather(rd, [row_ids, col]))
            out_v[pl.ds(c * CHUNK + g * 16, 16)] = acc
            return carry
        lax.fori_loop(0, GROUPS, group_body, 0)

    start(0, rs_a, rd_a, ss_a, sd_a)

    def pair_body(i, carry):
        c0 = 2 * i

        @pl.when(c0 + 1 < NCH)
        def _():
            start(c0 + 1, rs_b, rd_b, ss_b, sd_b)

        wait(rs_a, rd_a, ss_a, sd_a)
        compute(c0, rs_a, rd_a)

        @pl.when(c0 + 2 < NCH)
        def _():
            start(c0 + 2, rs_a, rd_a, ss_a, sd_a)

        @pl.when(c0 + 1 < NCH)
        def _():
            wait(rs_b, rd_b, ss_b, sd_b)
            compute(c0 + 1, rs_b, rd_b)

        return carry

    lax.fori_loop(0, (NCH + 1) // 2, pair_body, 0)
    pltpu.sync_copy(out_v, out_hbm.at[pl.ds(base, PER_W)])


@functools.cache
def _edge_scores():
    return pl.kernel(
        _edge_body,
        out_type=jax.ShapeDtypeStruct((E,), jnp.float32),
        mesh=plsc.VectorSubcoreMesh(
            core_axis_name="c", subcore_axis_name="s",
            num_cores=SC_CORES, num_subcores=SC_SUBCORES),
        compiler_params=pltpu.CompilerParams(
            needs_layout_passes=False, use_tc_tiling_on_sc=False),
        scratch_types=[
            pltpu.VMEM((PER_W,), jnp.int32),
            pltpu.VMEM((PER_W,), jnp.int32),
            pltpu.VMEM((CHUNK, N_OBJ), jnp.float32),
            pltpu.VMEM((CHUNK, N_OBJ), jnp.float32),
            pltpu.VMEM((CHUNK, N_OBJ), jnp.float32),
            pltpu.VMEM((CHUNK, N_OBJ), jnp.float32),
            pltpu.VMEM((PER_W,), jnp.float32),
            pltpu.SemaphoreType.DMA,
            pltpu.SemaphoreType.DMA,
            pltpu.SemaphoreType.DMA,
            pltpu.SemaphoreType.DMA,
        ],
    )


def kernel(node_features, node_hidden, edge_index,
           obj_W1, obj_b1, obj_W2, obj_b2, obj_W3, obj_b3,
           nc_W1, nc_b1, nc_W2, nc_b2, nc_W3, nc_b3):
    mean_h = _col_mean(node_hidden)
    obj_pred, node_pred = _mlps(
        node_features, node_hidden, mean_h,
        obj_W1, obj_b1.reshape(1, H), obj_W2, obj_b2.reshape(1, H),
        obj_W3, obj_b3.reshape(1, N_OBJ),
        nc_W1, nc_b1.reshape(1, H), nc_W2, nc_b2.reshape(1, H),
        nc_W3, nc_b3.reshape(1, N_CLS))
    src = edge_index[0].astype(jnp.int32)
    dst = edge_index[1].astype(jnp.int32)
    edge_pred = _edge_scores()(obj_pred, src, dst)
    return obj_pred, edge_pred, node_pred


# speedup vs baseline: 13.9280x; 3.3517x over previous
import functools

import jax
import jax.numpy as jnp
from jax import lax
from jax.experimental import pallas as pl
from jax.experimental.pallas import tpu as pltpu
from jax.experimental.pallas import tpu_sc as plsc

N = 10000
E = 320000
D_FEAT = 128
D_HID = 128
H = 256
N_OBJ = 32
N_CLS = 8

ROW_BLK = 1000

SC_CORES = 2
SC_SUBCORES = 16
NW = SC_CORES * SC_SUBCORES
PER_W = E // NW
CHUNK = 400
GROUPS = CHUNK // 16
NCH = PER_W // CHUNK


def _mean_body(hid_ref, out_ref):
    i = pl.program_id(0)

    @pl.when(i == 0)
    def _init():
        out_ref[...] = jnp.zeros_like(out_ref)

    out_ref[...] += jnp.sum(hid_ref[...], axis=0, keepdims=True)

    @pl.when(i == pl.num_programs(0) - 1)
    def _fin():
        out_ref[...] = out_ref[...] * (1.0 / N)


def _col_mean(node_hidden):
    return pl.pallas_call(
        _mean_body,
        grid=(N // ROW_BLK,),
        in_specs=[pl.BlockSpec((ROW_BLK, D_HID), lambda i: (i, 0))],
        out_specs=pl.BlockSpec((1, D_HID), lambda i: (0, 0)),
        out_shape=jax.ShapeDtypeStruct((1, D_HID), jnp.float32),
    )(node_hidden)


def _mlp_body(feat_ref, hid_ref, mean_ref,
              oW1, ob1, oW2, ob2, oW3, ob3,
              nW1, nb1, nW2, nb2, nW3, nb3,
              obj_ref, cls_ref):
    x = jnp.concatenate(
        [feat_ref[...], hid_ref[...],
         jnp.broadcast_to(mean_ref[...], (ROW_BLK, D_HID))], axis=1)
    h = jnp.dot(x, oW1[...], preferred_element_type=jnp.float32) + ob1[...]
    h = jnp.maximum(
        jnp.dot(h, oW2[...], preferred_element_type=jnp.float32) + ob2[...], 0.0)
    obj_ref[...] = jnp.maximum(
        jnp.dot(h, oW3[...], preferred_element_type=jnp.float32) + ob3[...], 0.0)
    g = jnp.dot(x, nW1[...], preferred_element_type=jnp.float32) + nb1[...]
    g = jnp.maximum(
        jnp.dot(g, nW2[...], preferred_element_type=jnp.float32) + nb2[...], 0.0)
    cls_ref[...] = jnp.dot(g, nW3[...], preferred_element_type=jnp.float32) + nb3[...]


def _full(shape):
    return pl.BlockSpec(shape, lambda i: tuple(0 for _ in shape))


def _mlps(node_features, node_hidden, mean_h,
          oW1, ob1, oW2, ob2, oW3, ob3,
          nW1, nb1, nW2, nb2, nW3, nb3):
    return pl.pallas_call(
        _mlp_body,
        grid=(N // ROW_BLK,),
        in_specs=[
            pl.BlockSpec((ROW_BLK, D_FEAT), lambda i: (i, 0)),
            pl.BlockSpec((ROW_BLK, D_HID), lambda i: (i, 0)),
            _full((1, D_HID)),
            _full((D_FEAT + 2 * D_HID, H)), _full((1, H)),
            _full((H, H)), _full((1, H)),
            _full((H, N_OBJ)), _full((1, N_OBJ)),
            _full((D_FEAT + 2 * D_HID, H)), _full((1, H)),
            _full((H, H)), _full((1, H)),
            _full((H, N_CLS)), _full((1, N_CLS)),
        ],
        out_specs=[
            pl.BlockSpec((ROW_BLK, N_OBJ), lambda i: (i, 0)),
            pl.BlockSpec((ROW_BLK, N_CLS), lambda i: (i, 0)),
        ],
        out_shape=[
            jax.ShapeDtypeStruct((N, N_OBJ), jnp.float32),
            jax.ShapeDtypeStruct((N, N_CLS), jnp.float32),
        ],
    )(node_features, node_hidden, mean_h,
      oW1, ob1, oW2, ob2, oW3, ob3,
      nW1, nb1, nW2, nb2, nW3, nb3)


def _edge_body(table_hbm, src_hbm, dst_hbm, out_hbm,
               idx_s, idx_d, rs_a, rd_a, rs_b, rd_b, out_v,
               ss_a, sd_a, ss_b, sd_b):
    wid = lax.axis_index("s") * SC_CORES + lax.axis_index("c")
    base = wid * PER_W
    pltpu.sync_copy(src_hbm.at[pl.ds(base, PER_W)], idx_s)
    pltpu.sync_copy(dst_hbm.at[pl.ds(base, PER_W)], idx_d)
    lane = lax.iota(jnp.int32, 16)

    def start(c, rs, rd, ss, sd):
        off = c * CHUNK
        pltpu.make_async_copy(
            table_hbm.at[idx_s.at[pl.ds(off, CHUNK)]], rs, ss).start()
        pltpu.make_async_copy(
            table_hbm.at[idx_d.at[pl.ds(off, CHUNK)]], rd, sd).start()

    def wait(rs, rd, ss, sd):
        pltpu.make_async_copy(
            table_hbm.at[idx_s.at[pl.ds(0, CHUNK)]], rs, ss).wait()
        pltpu.make_async_copy(
            table_hbm.at[idx_d.at[pl.ds(0, CHUNK)]], rd, sd).wait()

    def compute(c, rs, rd):
        def group_body(g, carry):
            row_ids = g * 16 + lane
            acc = jnp.zeros((16,), jnp.float32)
            for k in range(N_OBJ):
                col = jnp.bitwise_and(lane + k, N_OBJ - 1)
                acc = acc + (plsc.load_gather(rs, [row_ids, col]) *
                             plsc.load_gather(rd, [row_ids, col]))
            out_v[pl.ds(c * CHUNK + g * 16, 16)] = acc
            return carry
        lax.fori_loop(0, GROUPS, group_body, 0)

    start(0, rs_a, rd_a, ss_a, sd_a)

    def pair_body(i, carry):
        c0 = 2 * i

        @pl.when(c0 + 1 < NCH)
        def _():
            start(c0 + 1, rs_b, rd_b, ss_b, sd_b)

        wait(rs_a, rd_a, ss_a, sd_a)
        compute(c0, rs_a, rd_a)

        @pl.when(c0 + 2 < NCH)
        def _():
            start(c0 + 2, rs_a, rd_a, ss_a, sd_a)

        @pl.when(c0 + 1 < NCH)
        def _():
            wait(rs_b, rd_b, ss_b, sd_b)
            compute(c0 + 1, rs_b, rd_b)

        return carry

    lax.fori_loop(0, (NCH + 1) // 2, pair_body, 0)
    pltpu.sync_copy(out_v, out_hbm.at[pl.ds(base, PER_W)])


@functools.cache
def _edge_scores():
    return pl.kernel(
        _edge_body,
        out_type=jax.ShapeDtypeStruct((E,), jnp.float32),
        mesh=plsc.VectorSubcoreMesh(
            core_axis_name="c", subcore_axis_name="s",
            num_cores=SC_CORES, num_subcores=SC_SUBCORES),
        compiler_params=pltpu.CompilerParams(
            needs_layout_passes=False, use_tc_tiling_on_sc=False),
        scratch_types=[
            pltpu.VMEM((PER_W,), jnp.int32),
            pltpu.VMEM((PER_W,), jnp.int32),
            pltpu.VMEM((CHUNK, N_OBJ), jnp.float32),
            pltpu.VMEM((CHUNK, N_OBJ), jnp.float32),
            pltpu.VMEM((CHUNK, N_OBJ), jnp.float32),
            pltpu.VMEM((CHUNK, N_OBJ), jnp.float32),
            pltpu.VMEM((PER_W,), jnp.float32),
            pltpu.SemaphoreType.DMA,
            pltpu.SemaphoreType.DMA,
            pltpu.SemaphoreType.DMA,
            pltpu.SemaphoreType.DMA,
        ],
    )


def kernel(node_features, node_hidden, edge_index,
           obj_W1, obj_b1, obj_W2, obj_b2, obj_W3, obj_b3,
           nc_W1, nc_b1, nc_W2, nc_b2, nc_W3, nc_b3):
    mean_h = _col_mean(node_hidden)
    obj_pred, node_pred = _mlps(
        node_features, node_hidden, mean_h,
        obj_W1, obj_b1.reshape(1, H), obj_W2, obj_b2.reshape(1, H),
        obj_W3, obj_b3.reshape(1, N_OBJ),
        nc_W1, nc_b1.reshape(1, H), nc_W2, nc_b2.reshape(1, H),
        nc_W3, nc_b3.reshape(1, N_CLS))
    src = edge_index[0].astype(jnp.int32)
    dst = edge_index[1].astype(jnp.int32)
    edge_pred = _edge_scores()(obj_pred, src, dst)
    return obj_pred, edge_pred, node_pred
